# baseline (speedup 1.0000x reference)
import functools

import jax
import jax.numpy as jnp
import numpy as np
from jax.experimental import pallas as pl
from jax.experimental.pallas import tpu as pltpu

H = 56
W = 56
N = H * W
C = 720
K = 19
M = 10
MARGIN = 64
FP = MARGIN + N + MARGIN
BLK = 448
GRID = N // BLK

_BN_S = np.float32(1.0 / np.sqrt(1.0 + 1e-5))


def _interp_mat(s, t):
    ys = np.linspace(0.0, s - 1.0, t)
    y0 = np.minimum(np.floor(ys).astype(np.int64), s - 1)
    y1 = np.minimum(y0 + 1, s - 1)
    wy = (ys - y0).astype(np.float64)
    r = np.zeros((t, s), np.float64)
    r[np.arange(t), y0] += 1.0 - wy
    r[np.arange(t), y1] += wy
    return r


@functools.lru_cache(maxsize=None)
def _q_mat(s):
    r = _interp_mat(s, H)
    return jnp.asarray(np.kron(r, r).astype(np.float32))


def _assemble_body(f1, f2, f3, f4, q2, q3, q4, out):
    hp = jax.lax.Precision.HIGHEST
    r2 = jnp.dot(q2[:], f2[:], precision=hp, preferred_element_type=jnp.float32)
    r3 = jnp.dot(q3[:], f3[:], precision=hp, preferred_element_type=jnp.float32)
    r4 = jnp.dot(q4[:], f4[:], precision=hp, preferred_element_type=jnp.float32)
    feats = jnp.concatenate([f1[:], r2, r3, r4], axis=1).astype(jnp.bfloat16)
    out[pl.ds(0, MARGIN), :] = jnp.zeros((MARGIN, C), jnp.bfloat16)
    out[pl.ds(MARGIN + N, MARGIN), :] = jnp.zeros((MARGIN, C), jnp.bfloat16)
    out[pl.ds(MARGIN, N), :] = feats


def _conv3x3_body(fp, wr, g, bb, cb, out, w9s):
    @pl.when(pl.program_id(0) == 0)
    def _prep():
        wt = jnp.transpose(wr[:]).reshape(C, 9, C)
        for t in range(9):
            w9s[t] = wt[:, t, :]

    r0 = pl.program_id(0) * BLK
    acc = jnp.zeros((BLK, C), jnp.float32)
    iota = jax.lax.broadcasted_iota(jnp.int32, (BLK, 1), 0)
    for t in range(9):
        off = (t // 3 - 1) * W + (t % 3 - 1)
        align = (off // 8) * 8
        rem = off - align
        xs = fp[pl.ds(MARGIN + r0 + align, BLK + 8), :][rem:rem + BLK, :]
        dx = t % 3
        if dx == 0:
            keep = (iota + (off % W)) % W != W - 1
            xs = jnp.where(keep, xs, jnp.bfloat16(0))
        elif dx == 2:
            keep = (iota + (off % W)) % W != 0
            xs = jnp.where(keep, xs, jnp.bfloat16(0))
        acc += jnp.dot(xs, w9s[t], preferred_element_type=jnp.float32)
    s = g[:] * _BN_S
    b = cb[:] * s + bb[:]
    out[:] = jnp.maximum(acc * s + b, 0.0).astype(jnp.bfloat16)


def _proj_body(c1, p1w, p2w, p1b, pg, pbb, p2b, fng, fnb, pnt, mng, mnb, out):
    h = jnp.dot(c1[:], p1w[:], preferred_element_type=jnp.float32)
    s = pg[:] * _BN_S
    b = p1b[:] * s + pbb[:]
    h = jnp.maximum(h * s + b, 0.0).astype(jnp.bfloat16)
    c3 = jnp.dot(h, p2w[:], preferred_element_type=jnp.float32) + p2b[:]
    mu = jnp.mean(c3, axis=1, keepdims=True)
    xc = c3 - mu
    v = jnp.mean(xc * xc, axis=1, keepdims=True)
    xn = xc * jax.lax.rsqrt(v + 1e-5) * fng[:] + fnb[:]
    nrm = jnp.sqrt(jnp.sum(xn * xn, axis=1, keepdims=True))
    xl = (xn / jnp.maximum(nrm, 1e-12)).astype(jnp.bfloat16)
    pn = pnt[:]
    cn = jnp.sqrt(jnp.sum(pn * pn, axis=0, keepdims=True))
    pnb = (pn / jnp.maximum(cn, 1e-12)).astype(jnp.bfloat16)
    masks = jnp.dot(xl, pnb, preferred_element_type=jnp.float32)
    mm = masks[:, 0:K]
    for m in range(1, M):
        mm = jnp.maximum(mm, masks[:, m * K:(m + 1) * K])
    mu2 = jnp.mean(mm, axis=1, keepdims=True)
    yc = mm - mu2
    v2 = jnp.mean(yc * yc, axis=1, keepdims=True)
    out[:] = yc * jax.lax.rsqrt(v2 + 1e-5) * mng[:] + mnb[:]


def _btail_body(bf1, bw2t, bb2, bpt, out):
    bf = jnp.dot(bf1[:], bw2t[:], preferred_element_type=jnp.float32) + bb2[:]
    bps = jnp.sum(bpt[:], axis=0, keepdims=True)
    seg = None
    for m in range(M):
        sm = jnp.sum(bf * bps[:, m * K:(m + 1) * K], axis=1, keepdims=True)
        seg = sm if seg is None else jnp.maximum(seg, sm)
    out[:] = seg


def _full(shape):
    nd = len(shape)
    return pl.BlockSpec(shape, lambda *_: (0,) * nd)


def _rows(width):
    return pl.BlockSpec((BLK, width), lambda i: (i, 0))


def _vec(v):
    return v.reshape(1, -1).astype(jnp.float32)


def kernel(feat1, feat2, feat3, feat4, cw, cb, cg, cbb, p1w, p1b, pg, pbb,
           p2w, p2b, fng, fnb, mng, mnb, protos, bw1, bb1, bg, bbb, bw2,
           bb2, bprotos):
    f32 = jnp.float32
    bf16 = jnp.bfloat16

    f1t = jnp.transpose(feat1[0], (1, 2, 0)).reshape(N, 48)
    f2t = jnp.transpose(feat2[0], (1, 2, 0)).reshape(28 * 28, 96)
    f3t = jnp.transpose(feat3[0], (1, 2, 0)).reshape(14 * 14, 192)
    f4t = jnp.transpose(feat4[0], (1, 2, 0)).reshape(7 * 7, 384)

    w9c = cw.reshape(C, 9 * C).astype(bf16)
    w9b = bw1.reshape(C, 9 * C).astype(bf16)
    p1wt = p1w[:, :, 0, 0].T.astype(bf16)
    p2wt = p2w[:, :, 0, 0].T.astype(bf16)
    bw2t = bw2[:, :, 0, 0].T.astype(bf16)
    pnt = jnp.transpose(protos, (2, 1, 0)).reshape(C, M * K).astype(f32)
    bpt = jnp.transpose(bprotos, (2, 1, 0)).reshape(C, M * K).astype(f32)

    q2, q3, q4 = _q_mat(28), _q_mat(14), _q_mat(7)
    fpad = pl.pallas_call(
        _assemble_body,
        out_shape=jax.ShapeDtypeStruct((FP, C), bf16),
        in_specs=[_full((N, 48)), _full((28 * 28, 96)), _full((14 * 14, 192)),
                  _full((7 * 7, 384)), _full((N, 28 * 28)),
                  _full((N, 14 * 14)), _full((N, 7 * 7))],
        out_specs=_full((FP, C)),
    )(f1t, f2t, f3t, f4t, q2, q3, q4)

    conv = pl.pallas_call(
        _conv3x3_body,
        grid=(GRID,),
        out_shape=jax.ShapeDtypeStruct((N, C), bf16),
        in_specs=[_full((FP, C)), _full((C, 9 * C)), _full((1, C)),
                  _full((1, C)), _full((1, C))],
        out_specs=_rows(C),
        scratch_shapes=[pltpu.VMEM((9, C, C), bf16)],
    )
    c1 = conv(fpad, w9c, _vec(cg), _vec(cbb), _vec(cb))
    bf1 = conv(fpad, w9b, _vec(bg), _vec(bbb), _vec(bb1))

    seg = pl.pallas_call(
        _proj_body,
        grid=(GRID,),
        out_shape=jax.ShapeDtypeStruct((N, K), f32),
        in_specs=[_rows(C), _full((C, C)), _full((C, C)), _full((1, C)),
                  _full((1, C)), _full((1, C)), _full((1, C)), _full((1, C)),
                  _full((1, C)), _full((C, M * K)), _full((1, K)),
                  _full((1, K))],
        out_specs=_rows(K),
    )(c1, p1wt, p2wt, _vec(p1b), _vec(pg), _vec(pbb), _vec(p2b), _vec(fng),
      _vec(fnb), pnt, _vec(mng), _vec(mnb))

    bseg = pl.pallas_call(
        _btail_body,
        grid=(GRID,),
        out_shape=jax.ShapeDtypeStruct((N, 1), f32),
        in_specs=[_rows(C), _full((C, K)), _full((1, K)), _full((C, M * K))],
        out_specs=_rows(1),
    )(bf1, bw2t, _vec(bb2), bpt)

    out_seg = jnp.transpose(seg.reshape(H, W, K), (2, 0, 1))[None]
    out_b = bseg.reshape(1, H, W)
    return (out_seg, out_b)


# speedup vs baseline: 1.4512x; 1.4512x over previous
import functools

import jax
import jax.numpy as jnp
import numpy as np
from jax.experimental import pallas as pl
from jax.experimental.pallas import tpu as pltpu

H = 56
W = 56
N = H * W
C = 720
K = 19
M = 10
MARGIN = 64
FP = MARGIN + N + MARGIN
BLK = 448
GRID = N // BLK

_BN_S = np.float32(1.0 / np.sqrt(1.0 + 1e-5))


def _interp_mat(s, t):
    ys = np.linspace(0.0, s - 1.0, t)
    y0 = np.minimum(np.floor(ys).astype(np.int64), s - 1)
    y1 = np.minimum(y0 + 1, s - 1)
    wy = (ys - y0).astype(np.float64)
    r = np.zeros((t, s), np.float64)
    r[np.arange(t), y0] += 1.0 - wy
    r[np.arange(t), y1] += wy
    return r


@functools.lru_cache(maxsize=None)
def _q_mat(s):
    r = _interp_mat(s, H)
    return jnp.asarray(np.kron(r, r).astype(np.float32))


def _assemble_body(f1, f2, f3, f4, q2, q3, q4, out):
    hp = jax.lax.Precision.HIGHEST
    r2 = jnp.dot(q2[:], f2[:], precision=hp, preferred_element_type=jnp.float32)
    r3 = jnp.dot(q3[:], f3[:], precision=hp, preferred_element_type=jnp.float32)
    r4 = jnp.dot(q4[:], f4[:], precision=hp, preferred_element_type=jnp.float32)
    feats = jnp.concatenate([f1[:], r2, r3, r4], axis=1).astype(jnp.bfloat16)
    out[pl.ds(0, MARGIN), :] = jnp.zeros((MARGIN, C), jnp.bfloat16)
    out[pl.ds(MARGIN + N, MARGIN), :] = jnp.zeros((MARGIN, C), jnp.bfloat16)
    out[pl.ds(MARGIN, N), :] = feats


def _conv3x3_body(fp, w9s, g, bb, cb, out):
    r0 = pl.program_id(0) * BLK
    acc = jnp.zeros((BLK, C), jnp.float32)
    iota = jax.lax.broadcasted_iota(jnp.int32, (BLK, 1), 0)
    for t in range(9):
        off = (t // 3 - 1) * W + (t % 3 - 1)
        align = (off // 8) * 8
        rem = off - align
        xs = fp[pl.ds(MARGIN + r0 + align, BLK + 8), :][rem:rem + BLK, :]
        dx = t % 3
        if dx == 0:
            keep = (iota + (off % W)) % W != W - 1
            xs = jnp.where(keep, xs, jnp.bfloat16(0))
        elif dx == 2:
            keep = (iota + (off % W)) % W != 0
            xs = jnp.where(keep, xs, jnp.bfloat16(0))
        acc += jnp.dot(xs, w9s[t], preferred_element_type=jnp.float32)
    s = g[:] * _BN_S
    b = cb[:] * s + bb[:]
    out[:] = jnp.maximum(acc * s + b, 0.0).astype(jnp.bfloat16)


def _proj_body(c1, p1w, p2w, p1b, pg, pbb, p2b, fng, fnb, pnt, mng, mnb, out):
    h = jnp.dot(c1[:], p1w[:], preferred_element_type=jnp.float32)
    s = pg[:] * _BN_S
    b = p1b[:] * s + pbb[:]
    h = jnp.maximum(h * s + b, 0.0).astype(jnp.bfloat16)
    c3 = jnp.dot(h, p2w[:], preferred_element_type=jnp.float32) + p2b[:]
    mu = jnp.mean(c3, axis=1, keepdims=True)
    xc = c3 - mu
    v = jnp.mean(xc * xc, axis=1, keepdims=True)
    xn = xc * jax.lax.rsqrt(v + 1e-5) * fng[:] + fnb[:]
    nrm = jnp.sqrt(jnp.sum(xn * xn, axis=1, keepdims=True))
    xl = (xn / jnp.maximum(nrm, 1e-12)).astype(jnp.bfloat16)
    pn = pnt[:]
    cn = jnp.sqrt(jnp.sum(pn * pn, axis=0, keepdims=True))
    pnb = (pn / jnp.maximum(cn, 1e-12)).astype(jnp.bfloat16)
    masks = jnp.dot(xl, pnb, preferred_element_type=jnp.float32)
    mm = masks[:, 0:K]
    for m in range(1, M):
        mm = jnp.maximum(mm, masks[:, m * K:(m + 1) * K])
    mu2 = jnp.mean(mm, axis=1, keepdims=True)
    yc = mm - mu2
    v2 = jnp.mean(yc * yc, axis=1, keepdims=True)
    out[:] = yc * jax.lax.rsqrt(v2 + 1e-5) * mng[:] + mnb[:]


def _btail_body(bf1, bw2t, bb2, bpt, out):
    bf = jnp.dot(bf1[:], bw2t[:], preferred_element_type=jnp.float32) + bb2[:]
    bps = jnp.sum(bpt[:], axis=0, keepdims=True)
    seg = None
    for m in range(M):
        sm = jnp.sum(bf * bps[:, m * K:(m + 1) * K], axis=1, keepdims=True)
        seg = sm if seg is None else jnp.maximum(seg, sm)
    out[:] = seg


def _full(shape):
    nd = len(shape)
    return pl.BlockSpec(shape, lambda *_: (0,) * nd)


def _rows(width):
    return pl.BlockSpec((BLK, width), lambda i: (i, 0))


def _vec(v):
    return v.reshape(1, -1).astype(jnp.float32)


def kernel(feat1, feat2, feat3, feat4, cw, cb, cg, cbb, p1w, p1b, pg, pbb,
           p2w, p2b, fng, fnb, mng, mnb, protos, bw1, bb1, bg, bbb, bw2,
           bb2, bprotos):
    f32 = jnp.float32
    bf16 = jnp.bfloat16

    f1t = jnp.transpose(feat1[0], (1, 2, 0)).reshape(N, 48)
    f2t = jnp.transpose(feat2[0], (1, 2, 0)).reshape(28 * 28, 96)
    f3t = jnp.transpose(feat3[0], (1, 2, 0)).reshape(14 * 14, 192)
    f4t = jnp.transpose(feat4[0], (1, 2, 0)).reshape(7 * 7, 384)

    w9c = jnp.transpose(cw.astype(bf16), (2, 3, 1, 0)).reshape(9, C, C)
    w9b = jnp.transpose(bw1.astype(bf16), (2, 3, 1, 0)).reshape(9, C, C)
    p1wt = p1w[:, :, 0, 0].T.astype(bf16)
    p2wt = p2w[:, :, 0, 0].T.astype(bf16)
    bw2t = bw2[:, :, 0, 0].T.astype(bf16)
    pnt = jnp.transpose(protos, (2, 1, 0)).reshape(C, M * K).astype(f32)
    bpt = jnp.transpose(bprotos, (2, 1, 0)).reshape(C, M * K).astype(f32)

    q2, q3, q4 = _q_mat(28), _q_mat(14), _q_mat(7)
    fpad = pl.pallas_call(
        _assemble_body,
        out_shape=jax.ShapeDtypeStruct((FP, C), bf16),
        in_specs=[_full((N, 48)), _full((28 * 28, 96)), _full((14 * 14, 192)),
                  _full((7 * 7, 384)), _full((N, 28 * 28)),
                  _full((N, 14 * 14)), _full((N, 7 * 7))],
        out_specs=_full((FP, C)),
    )(f1t, f2t, f3t, f4t, q2, q3, q4)

    conv = pl.pallas_call(
        _conv3x3_body,
        grid=(GRID,),
        out_shape=jax.ShapeDtypeStruct((N, C), bf16),
        in_specs=[_full((FP, C)), _full((9, C, C)), _full((1, C)),
                  _full((1, C)), _full((1, C))],
        out_specs=_rows(C),
        compiler_params=pltpu.CompilerParams(
            dimension_semantics=("parallel",)),
    )
    c1 = conv(fpad, w9c, _vec(cg), _vec(cbb), _vec(cb))
    bf1 = conv(fpad, w9b, _vec(bg), _vec(bbb), _vec(bb1))

    seg = pl.pallas_call(
        _proj_body,
        grid=(GRID,),
        out_shape=jax.ShapeDtypeStruct((N, K), f32),
        in_specs=[_rows(C), _full((C, C)), _full((C, C)), _full((1, C)),
                  _full((1, C)), _full((1, C)), _full((1, C)), _full((1, C)),
                  _full((1, C)), _full((C, M * K)), _full((1, K)),
                  _full((1, K))],
        out_specs=_rows(K),
        compiler_params=pltpu.CompilerParams(
            dimension_semantics=("parallel",)),
    )(c1, p1wt, p2wt, _vec(p1b), _vec(pg), _vec(pbb), _vec(p2b), _vec(fng),
      _vec(fnb), pnt, _vec(mng), _vec(mnb))

    bseg = pl.pallas_call(
        _btail_body,
        grid=(GRID,),
        out_shape=jax.ShapeDtypeStruct((N, 1), f32),
        in_specs=[_rows(C), _full((C, K)), _full((1, K)), _full((C, M * K))],
        out_specs=_rows(1),
        compiler_params=pltpu.CompilerParams(
            dimension_semantics=("parallel",)),
    )(bf1, bw2t, _vec(bb2), bpt)

    out_seg = jnp.transpose(seg.reshape(H, W, K), (2, 0, 1))[None]
    out_b = bseg.reshape(1, H, W)
    return (out_seg, out_b)


# speedup vs baseline: 1.5990x; 1.1018x over previous
import functools

import jax
import jax.numpy as jnp
import numpy as np
from jax.experimental import pallas as pl
from jax.experimental.pallas import tpu as pltpu

H = 56
W = 56
N = H * W
C = 720
K = 19
M = 10
MARGIN = 64
FP = MARGIN + N + MARGIN
BLK = 448
GRID = N // BLK

_BN_S = np.float32(1.0 / np.sqrt(1.0 + 1e-5))


def _interp_mat(s, t):
    ys = np.linspace(0.0, s - 1.0, t)
    y0 = np.minimum(np.floor(ys).astype(np.int64), s - 1)
    y1 = np.minimum(y0 + 1, s - 1)
    wy = (ys - y0).astype(np.float64)
    r = np.zeros((t, s), np.float64)
    r[np.arange(t), y0] += 1.0 - wy
    r[np.arange(t), y1] += wy
    return r


@functools.lru_cache(maxsize=None)
def _q_mat(s):
    r = _interp_mat(s, H)
    return jnp.asarray(np.kron(r, r).astype(np.float32)).astype(jnp.bfloat16)


def _assemble_body(f1, f2, f3, f4, q2, q3, q4, out):
    r2 = jnp.dot(q2[:], f2[:], preferred_element_type=jnp.float32)
    r3 = jnp.dot(q3[:], f3[:], preferred_element_type=jnp.float32)
    r4 = jnp.dot(q4[:], f4[:], preferred_element_type=jnp.float32)
    feats = jnp.concatenate([f1[:].astype(jnp.float32), r2, r3, r4],
                            axis=1).astype(jnp.bfloat16)
    out[pl.ds(0, MARGIN), :] = jnp.zeros((MARGIN, C), jnp.bfloat16)
    out[pl.ds(MARGIN + N, MARGIN), :] = jnp.zeros((MARGIN, C), jnp.bfloat16)
    out[pl.ds(MARGIN, N), :] = feats


def _conv3x3_body(fp, w9s, g, bb, cb, out):
    r0 = pl.program_id(0) * BLK
    acc = jnp.zeros((BLK, C), jnp.float32)
    iota = jax.lax.broadcasted_iota(jnp.int32, (BLK, 1), 0)
    for t in range(9):
        off = (t // 3 - 1) * W + (t % 3 - 1)
        align = (off // 8) * 8
        rem = off - align
        xs = fp[pl.ds(MARGIN + r0 + align, BLK + 8), :][rem:rem + BLK, :]
        dx = t % 3
        if dx == 0:
            keep = (iota + (off % W)) % W != W - 1
            xs = jnp.where(keep, xs, jnp.bfloat16(0))
        elif dx == 2:
            keep = (iota + (off % W)) % W != 0
            xs = jnp.where(keep, xs, jnp.bfloat16(0))
        acc += jnp.dot(xs, w9s[t], preferred_element_type=jnp.float32)
    s = g[:] * _BN_S
    b = cb[:] * s + bb[:]
    out[:] = jnp.maximum(acc * s + b, 0.0).astype(jnp.bfloat16)


def _proj_body(c1, p1w, p2w, p1b, pg, pbb, p2b, fng, fnb, pnt, mng, mnb, out):
    h = jnp.dot(c1[:], p1w[:], preferred_element_type=jnp.float32)
    s = pg[:] * _BN_S
    b = p1b[:] * s + pbb[:]
    h = jnp.maximum(h * s + b, 0.0).astype(jnp.bfloat16)
    c3 = jnp.dot(h, p2w[:], preferred_element_type=jnp.float32) + p2b[:]
    mu = jnp.mean(c3, axis=1, keepdims=True)
    xc = c3 - mu
    v = jnp.mean(xc * xc, axis=1, keepdims=True)
    xn = xc * jax.lax.rsqrt(v + 1e-5) * fng[:] + fnb[:]
    nrm = jnp.sqrt(jnp.sum(xn * xn, axis=1, keepdims=True))
    xl = (xn / jnp.maximum(nrm, 1e-12)).astype(jnp.bfloat16)
    pn = pnt[:]
    cn = jnp.sqrt(jnp.sum(pn * pn, axis=0, keepdims=True))
    pnb = (pn / jnp.maximum(cn, 1e-12)).astype(jnp.bfloat16)
    masks = jnp.dot(xl, pnb, preferred_element_type=jnp.float32)
    mm = masks[:, 0:K]
    for m in range(1, M):
        mm = jnp.maximum(mm, masks[:, m * K:(m + 1) * K])
    mu2 = jnp.mean(mm, axis=1, keepdims=True)
    yc = mm - mu2
    v2 = jnp.mean(yc * yc, axis=1, keepdims=True)
    out[:] = yc * jax.lax.rsqrt(v2 + 1e-5) * mng[:] + mnb[:]


def _btail_body(bf1, bw2t, bb2, bpt, out):
    bf = jnp.dot(bf1[:], bw2t[:], preferred_element_type=jnp.float32) + bb2[:]
    bps = jnp.sum(bpt[:], axis=0, keepdims=True)
    seg = None
    for m in range(M):
        sm = jnp.sum(bf * bps[:, m * K:(m + 1) * K], axis=1, keepdims=True)
        seg = sm if seg is None else jnp.maximum(seg, sm)
    out[:] = seg


def _full(shape):
    nd = len(shape)
    return pl.BlockSpec(shape, lambda *_: (0,) * nd)


def _rows(width):
    return pl.BlockSpec((BLK, width), lambda i: (i, 0))


def _vec(v):
    return v.reshape(1, -1).astype(jnp.float32)


def kernel(feat1, feat2, feat3, feat4, cw, cb, cg, cbb, p1w, p1b, pg, pbb,
           p2w, p2b, fng, fnb, mng, mnb, protos, bw1, bb1, bg, bbb, bw2,
           bb2, bprotos):
    f32 = jnp.float32
    bf16 = jnp.bfloat16

    f1t = jnp.transpose(feat1[0], (1, 2, 0)).reshape(N, 48).astype(bf16)
    f2t = jnp.transpose(feat2[0], (1, 2, 0)).reshape(28 * 28, 96).astype(bf16)
    f3t = jnp.transpose(feat3[0], (1, 2, 0)).reshape(14 * 14, 192).astype(bf16)
    f4t = jnp.transpose(feat4[0], (1, 2, 0)).reshape(7 * 7, 384).astype(bf16)

    cwh, bw1h = jax.lax.optimization_barrier((cw.astype(bf16),
                                              bw1.astype(bf16)))
    w9c = jnp.transpose(cwh, (2, 3, 1, 0)).reshape(9, C, C)
    w9b = jnp.transpose(bw1h, (2, 3, 1, 0)).reshape(9, C, C)
    p1wt = p1w[:, :, 0, 0].T.astype(bf16)
    p2wt = p2w[:, :, 0, 0].T.astype(bf16)
    bw2t = bw2[:, :, 0, 0].T.astype(bf16)
    pnt = jnp.transpose(protos, (2, 1, 0)).reshape(C, M * K).astype(f32)
    bpt = jnp.transpose(bprotos, (2, 1, 0)).reshape(C, M * K).astype(f32)

    q2, q3, q4 = _q_mat(28), _q_mat(14), _q_mat(7)
    fpad = pl.pallas_call(
        _assemble_body,
        out_shape=jax.ShapeDtypeStruct((FP, C), bf16),
        in_specs=[_full((N, 48)), _full((28 * 28, 96)), _full((14 * 14, 192)),
                  _full((7 * 7, 384)), _full((N, 28 * 28)),
                  _full((N, 14 * 14)), _full((N, 7 * 7))],
        out_specs=_full((FP, C)),
    )(f1t, f2t, f3t, f4t, q2, q3, q4)

    conv = pl.pallas_call(
        _conv3x3_body,
        grid=(GRID,),
        out_shape=jax.ShapeDtypeStruct((N, C), bf16),
        in_specs=[_full((FP, C)), _full((9, C, C)), _full((1, C)),
                  _full((1, C)), _full((1, C))],
        out_specs=_rows(C),
        compiler_params=pltpu.CompilerParams(
            dimension_semantics=("parallel",)),
    )
    c1 = conv(fpad, w9c, _vec(cg), _vec(cbb), _vec(cb))
    bf1 = conv(fpad, w9b, _vec(bg), _vec(bbb), _vec(bb1))

    seg = pl.pallas_call(
        _proj_body,
        grid=(GRID,),
        out_shape=jax.ShapeDtypeStruct((N, K), f32),
        in_specs=[_rows(C), _full((C, C)), _full((C, C)), _full((1, C)),
                  _full((1, C)), _full((1, C)), _full((1, C)), _full((1, C)),
                  _full((1, C)), _full((C, M * K)), _full((1, K)),
                  _full((1, K))],
        out_specs=_rows(K),
        compiler_params=pltpu.CompilerParams(
            dimension_semantics=("parallel",)),
    )(c1, p1wt, p2wt, _vec(p1b), _vec(pg), _vec(pbb), _vec(p2b), _vec(fng),
      _vec(fnb), pnt, _vec(mng), _vec(mnb))

    bseg = pl.pallas_call(
        _btail_body,
        grid=(GRID,),
        out_shape=jax.ShapeDtypeStruct((N, 1), f32),
        in_specs=[_rows(C), _full((C, K)), _full((1, K)), _full((C, M * K))],
        out_specs=_rows(1),
        compiler_params=pltpu.CompilerParams(
            dimension_semantics=("parallel",)),
    )(bf1, bw2t, _vec(bb2), bpt)

    out_seg = jnp.transpose(seg.reshape(H, W, K), (2, 0, 1))[None]
    out_b = bseg.reshape(1, H, W)
    return (out_seg, out_b)
